# Initial kernel scaffold; baseline (speedup 1.0000x reference)
import jax
import jax.numpy as jnp
from jax.experimental import pallas as pl


def kernel(x, edge_index, edge_weight, mask_teams, W1_0, b1_0, g1_0, bt1_0, W2_0, b2_0, g2_0, bt2_0, eps_0, W1_1, b1_1, g1_1, bt1_1, W2_1, b2_1, g2_1, bt2_1, eps_1):
    raise NotImplementedError("write your pallas kernel here")


# speedup vs baseline: 3.5476x; 3.5476x over previous
import functools

import jax
import jax.numpy as jnp
from jax import lax
from jax.experimental import pallas as pl
from jax.experimental.pallas import tpu as pltpu
from jax.experimental.pallas import tpu_sc as plsc

N_NODES = 10000
N_EDGES = 320000
D = 128
BN_EPS = 1e-5

NC = 2
NS = 16
NW = NC * NS
EPT = N_EDGES // NW
EC = 200
NCHUNK = EPT // EC
N_PAD = 10240
RPT = N_PAD // NS

_sc_mesh = plsc.VectorSubcoreMesh(core_axis_name="c", subcore_axis_name="s")


def _agg_body(x_hbm, src_hbm, dst_hbm, ew_hbm, out_hbm, agg_sh, src_v, dst_v,
              ew_v, rows_v, sem):
    cid = lax.axis_index("c")
    sid = lax.axis_index("s")

    zeros = jnp.zeros((16,), jnp.float32)

    def _zbody(i, _):
        r = i // 8
        col = (i % 8) * 16
        rows_v[r, pl.ds(col, 16)] = zeros
        return 0

    lax.fori_loop(0, EC * 8, _zbody, 0)

    base_r = sid * RPT
    for off in range(0, RPT, EC):
        ln = min(EC, RPT - off)
        pltpu.sync_copy(rows_v.at[pl.ds(0, ln)],
                        agg_sh.at[pl.ds(base_r + off, ln)])
    plsc.subcore_barrier()

    base_e0 = (cid * NS + sid) * EPT

    def _chunk(k, _):
        be = base_e0 + k * EC
        pltpu.sync_copy(src_hbm.at[pl.ds(be, EC)], src_v)
        pltpu.sync_copy(dst_hbm.at[pl.ds(be, EC)], dst_v)
        pltpu.sync_copy(ew_hbm.at[pl.ds(be * 16, EC * 16)], ew_v)
        pltpu.async_copy(x_hbm.at[src_v], rows_v, sem).wait()

        def _scale(e, _):
            w = ew_v[pl.ds(e * 16, 16)]
            for j in range(8):
                rows_v[e, pl.ds(j * 16, 16)] = rows_v[e, pl.ds(j * 16, 16)] * w
            return 0

        lax.fori_loop(0, EC, _scale, 0)
        pltpu.sync_copy(rows_v, agg_sh.at[dst_v], add=True)
        return 0

    lax.fori_loop(0, NCHUNK, _chunk, 0)
    plsc.subcore_barrier()
    pltpu.sync_copy(agg_sh.at[pl.ds(base_r, RPT)],
                    out_hbm.at[cid, pl.ds(base_r, RPT)])


_sc_agg = pl.kernel(
    _agg_body,
    mesh=_sc_mesh,
    out_type=jax.ShapeDtypeStruct((NC, N_PAD, D), jnp.float32),
    scratch_types=[
        pltpu.VMEM_SHARED((N_PAD, D), jnp.float32),
        pltpu.VMEM((EC,), jnp.int32),
        pltpu.VMEM((EC,), jnp.int32),
        pltpu.VMEM((EC * 16,), jnp.float32),
        pltpu.VMEM((EC, D), jnp.float32),
        pltpu.SemaphoreType.DMA,
    ],
)

MW = 25
MB = 40


def _mask_body(h_hbm, mask_hbm, out_hbm, idx_v, rows_v, sem):
    cid = lax.axis_index("c")
    sid = lax.axis_index("s")
    wid = sid * NC + cid

    @pl.when(wid < MW)
    def _():
        base = wid * MB
        pltpu.sync_copy(mask_hbm.at[pl.ds(base, MB)], idx_v)
        pltpu.async_copy(h_hbm.at[idx_v], rows_v, sem).wait()
        pltpu.sync_copy(rows_v, out_hbm.at[pl.ds(base, MB)])


_sc_mask = pl.kernel(
    _mask_body,
    mesh=_sc_mesh,
    out_type=jax.ShapeDtypeStruct((1000, D), jnp.float32),
    scratch_types=[
        pltpu.VMEM((MB,), jnp.int32),
        pltpu.VMEM((MB, D), jnp.float32),
        pltpu.SemaphoreType.DMA,
    ],
)


def _mlp_body(x_ref, agg_ref, eps_ref, w1_ref, b1_ref, g1_ref, bt1_ref,
              w2_ref, b2_ref, g2_ref, bt2_ref, out_ref):
    eps = eps_ref[0, 0]
    a = agg_ref[...]
    h = x_ref[...] * (1.0 + eps) + a[0, :N_NODES] + a[1, :N_NODES]
    h = jnp.dot(h, w1_ref[...], preferred_element_type=jnp.float32)
    h = h + b1_ref[...]
    mu = jnp.mean(h, axis=0, keepdims=True)
    var = jnp.mean((h - mu) ** 2, axis=0, keepdims=True)
    h = (h - mu) * lax.rsqrt(var + BN_EPS) * g1_ref[...] + bt1_ref[...]
    h = jnp.maximum(h, 0.0)
    h = jnp.dot(h, w2_ref[...], preferred_element_type=jnp.float32)
    h = h + b2_ref[...]
    mu = jnp.mean(h, axis=0, keepdims=True)
    var = jnp.mean((h - mu) ** 2, axis=0, keepdims=True)
    h = (h - mu) * lax.rsqrt(var + BN_EPS) * g2_ref[...] + bt2_ref[...]
    out_ref[...] = jnp.maximum(h, 0.0)


def _tc_mlp(x, agg, eps, w1, b1, g1, bt1, w2, b2, g2, bt2):
    return pl.pallas_call(
        _mlp_body,
        out_shape=jax.ShapeDtypeStruct((N_NODES, D), jnp.float32),
    )(x, agg, jnp.reshape(eps, (1, 1)),
      w1, jnp.reshape(b1, (1, D)), jnp.reshape(g1, (1, D)),
      jnp.reshape(bt1, (1, D)),
      w2, jnp.reshape(b2, (1, D)), jnp.reshape(g2, (1, D)),
      jnp.reshape(bt2, (1, D)))


def kernel(x, edge_index, edge_weight, mask_teams,
           W1_0, b1_0, g1_0, bt1_0, W2_0, b2_0, g2_0, bt2_0, eps_0,
           W1_1, b1_1, g1_1, bt1_1, W2_1, b2_1, g2_1, bt2_1, eps_1):
    src = edge_index[0]
    dst = edge_index[1]
    ew16 = jnp.reshape(jnp.broadcast_to(edge_weight[:, None], (N_EDGES, 16)),
                       (N_EDGES * 16,))
    agg = _sc_agg(x, src, dst, ew16)
    h = _tc_mlp(x, agg, eps_0, W1_0, b1_0, g1_0, bt1_0,
                W2_0, b2_0, g2_0, bt2_0)
    agg = _sc_agg(h, src, dst, ew16)
    h = _tc_mlp(h, agg, eps_1, W1_1, b1_1, g1_1, bt1_1,
                W2_1, b2_1, g2_1, bt2_1)
    return _sc_mask(h, mask_teams)
